# baseline (speedup 1.0000x reference)
import jax
import jax.numpy as jnp
from jax.experimental import pallas as pl
from jax.experimental.pallas import tpu as pltpu

_K = 512
_NUM_ITERS = 4
_TN = 2048


def _kmeans_body(xf_ref, xtb_ref, cidx_ref, ridx_ref, centers_ref, assign_ref):
    N = xf_ref.shape[1]
    da = xf_ref.shape[2]
    D = da - 1
    nt = N // _TN
    cidx = cidx_ref[0]
    ridx_all = ridx_ref[0]

    iota_k = jax.lax.broadcasted_iota(jnp.int32, (_K, _TN), 0)
    iota_n0 = jax.lax.broadcasted_iota(jnp.int32, (_K, _TN), 1)

    def hdot(oh, t, acc):
        xtf = xf_ref[0, pl.ds(t * _TN, _TN), :]
        return acc + jax.lax.dot_general(
            oh, xtf, (((1,), (0,)), ((), ())),
            preferred_element_type=jnp.float32,
            precision=jax.lax.Precision.HIGHEST)

    centers = jnp.zeros((_K, da), jnp.float32)
    for t in range(nt):
        centers = hdot((cidx == iota_n0 + t * _TN).astype(jnp.float32),
                       t, centers)
    centers = centers[:, :D]

    for i in range(_NUM_ITERS):
        c2 = jnp.sum(centers * centers, axis=1, keepdims=True)
        cb = centers.astype(jnp.bfloat16)
        ridx = ridx_all[:, i:i + 1]
        sums = jnp.zeros((_K, da), jnp.float32)
        for t in range(nt):
            xtt = xtb_ref[0, :, pl.ds(t * _TN, _TN)]
            xct = jax.lax.dot_general(
                cb, xtt.astype(jnp.bfloat16), (((1,), (0,)), ((), ())),
                preferred_element_type=jnp.float32)
            d2 = c2 - 2.0 * xct
            m = jnp.min(d2, axis=0, keepdims=True)
            masked = jnp.where(d2 == m, iota_k, _K)
            assign = jnp.min(masked, axis=0, keepdims=True)
            if i == _NUM_ITERS - 1:
                assign_ref[0, :, pl.ds(t * _TN, _TN)] = assign
            oh = (iota_k == assign).astype(jnp.float32)
            sums = hdot(oh, t, sums)
        counts = sums[:, D:da]

        def mk_repl():
            acc = jnp.zeros((_K, da), jnp.float32)
            for t in range(nt):
                acc = hdot((ridx == iota_n0 + t * _TN).astype(jnp.float32),
                           t, acc)
            return acc[:, :D]

        repl = jax.lax.cond(jnp.any(counts == 0.0), mk_repl,
                            lambda: jnp.zeros((_K, D), jnp.float32))
        centers = jnp.where(counts == 0.0, repl,
                            sums[:, :D] / jnp.maximum(counts, 1.0))
    centers_ref[0] = centers


def kernel(x):
    B, N, D = x.shape
    key = jax.random.key(42)
    k_init, key = jax.random.split(key)
    random_order = jnp.argsort(jax.random.uniform(k_init, (B, N)), axis=1)
    cidx = random_order[:, :_K].astype(jnp.int32)[:, :, None]
    ridx = jnp.stack(
        [jax.random.randint(jax.random.fold_in(key, i), (B, _K), 0, N)
         for i in range(_NUM_ITERS)], axis=-1).astype(jnp.int32)
    x_aug = jnp.concatenate(
        [x, jnp.ones((B, N, 1), jnp.float32)], axis=2)
    xtb = jnp.swapaxes(x, 1, 2)
    centers, assign = pl.pallas_call(
        _kmeans_body,
        grid=(B,),
        compiler_params=pltpu.CompilerParams(
            dimension_semantics=("parallel",)),
        in_specs=[
            pl.BlockSpec((1, N, D + 1), lambda b: (b, 0, 0)),
            pl.BlockSpec((1, D, N), lambda b: (b, 0, 0)),
            pl.BlockSpec((1, _K, 1), lambda b: (b, 0, 0)),
            pl.BlockSpec((1, _K, _NUM_ITERS), lambda b: (b, 0, 0)),
        ],
        out_specs=[
            pl.BlockSpec((1, _K, D), lambda b: (b, 0, 0)),
            pl.BlockSpec((1, 1, N), lambda b: (b, 0, 0)),
        ],
        out_shape=[
            jax.ShapeDtypeStruct((B, _K, D), jnp.float32),
            jax.ShapeDtypeStruct((B, 1, N), jnp.int32),
        ],
    )(x_aug, xtb, cidx, ridx)
    return centers, assign.reshape(B, N)
